# baseline (speedup 1.0000x reference)
import functools

import jax
import jax.numpy as jnp
from jax import lax
from jax.experimental import pallas as pl
from jax.experimental.pallas import tpu as pltpu
from jax.experimental.pallas import tpu_sc as plsc

_NC = 2
_NS = 16
_C = 80
_EW = 16


def _sc_agg(NP, DH, NCH, NPT, DE):
  mesh = plsc.VectorSubcoreMesh(core_axis_name="c", subcore_axis_name="s")
  G = NCH // 2

  @functools.partial(
      pl.kernel,
      mesh=mesh,
      compiler_params=pltpu.CompilerParams(use_tc_tiling_on_sc=False),
      out_type=[
          jax.ShapeDtypeStruct((_NC, NP, DH), jnp.float32),
          jax.ShapeDtypeStruct((_NC, NP, _EW), jnp.float32),
      ],
      scratch_types=[
          pltpu.VMEM((NCH * _C,), jnp.int32),
          pltpu.VMEM((NCH, _C), jnp.int32),
          pltpu.VMEM((_C, DH), jnp.float32),
          pltpu.VMEM((_C, DH), jnp.float32),
          pltpu.VMEM((_C, _EW), jnp.float32),
          pltpu.VMEM((_C, _EW), jnp.float32),
          pltpu.VMEM_SHARED((NP, DH), jnp.float32),
          pltpu.VMEM_SHARED((NP, _EW), jnp.float32),
          pltpu.SemaphoreType.DMA,
          pltpu.SemaphoreType.DMA,
          pltpu.SemaphoreType.DMA,
          pltpu.SemaphoreType.DMA,
      ],
  )
  def sc_agg(src_hbm, dst_hbm, ea_hbm, ones_hbm, h_hbm, zh_hbm, ze_hbm,
             outh_hbm, oute_hbm,
             idx_s, idx_d, rows_v0, rows_v1, ea_v0, ea_v1,
             acc_h, acc_e, semA, semB, semC, semD):
    c = lax.axis_index("c")
    s = lax.axis_index("s")
    r0 = s * NPT
    hc = h_hbm.at[c]
    pltpu.sync_copy(zh_hbm, acc_h.at[pl.ds(r0, NPT)])
    pltpu.sync_copy(ze_hbm, acc_e.at[pl.ds(r0, NPT)])
    pltpu.sync_copy(src_hbm.at[1].at[pl.ds(s * NCH * _C, NCH * _C)], idx_s)
    pltpu.sync_copy(dst_hbm.at[s], idx_d)
    plsc.subcore_barrier()

    pltpu.async_copy(hc.at[idx_s.at[pl.ds(0, _C)]], rows_v0, semA)

    eb = s * NCH * _C

    @pl.when(c == 0)
    def _():
      pltpu.async_copy(ea_hbm.at[pl.ds(eb, _C)], ea_v0, semC)

    @pl.when(c == 1)
    def _():
      pltpu.sync_copy(ones_hbm, ea_v0)
      pltpu.sync_copy(ones_hbm, ea_v1)

    zh80 = zh_hbm.at[pl.ds(0, _C)]
    ze80 = ze_hbm.at[pl.ds(0, _C)]

    def body(g, carry):
      j0 = 2 * g
      j1 = j0 + 1
      pltpu.async_copy(hc.at[idx_s.at[pl.ds(j1 * _C, _C)]], rows_v1, semB)

      @pl.when(c == 0)
      def _():
        pltpu.async_copy(ea_hbm.at[pl.ds(eb + j1 * _C, _C)], ea_v1, semD)

      pltpu.make_async_copy(zh80, rows_v0, semA).wait()
      pltpu.sync_copy(rows_v0, acc_h.at[idx_d.at[j0]], add=True)

      @pl.when(c == 0)
      def _():
        pltpu.make_async_copy(ze80, ea_v0, semC).wait()

      pltpu.sync_copy(ea_v0, acc_e.at[idx_d.at[j0]], add=True)

      @pl.when(g < G - 1)
      def _():
        pltpu.async_copy(hc.at[idx_s.at[pl.ds((j0 + 2) * _C, _C)]], rows_v0,
                         semA)

      @pl.when(jnp.logical_and(g < G - 1, c == 0))
      def _():
        pltpu.async_copy(ea_hbm.at[pl.ds(eb + (j0 + 2) * _C, _C)], ea_v0, semC)

      pltpu.make_async_copy(zh80, rows_v1, semB).wait()
      pltpu.sync_copy(rows_v1, acc_h.at[idx_d.at[j1]], add=True)

      @pl.when(c == 0)
      def _():
        pltpu.make_async_copy(ze80, ea_v1, semD).wait()

      pltpu.sync_copy(ea_v1, acc_e.at[idx_d.at[j1]], add=True)
      return carry

    lax.fori_loop(0, G, body, 0)
    plsc.subcore_barrier()
    pltpu.sync_copy(acc_h.at[pl.ds(r0, NPT)], outh_hbm.at[c].at[pl.ds(r0, NPT)])
    pltpu.sync_copy(acc_e.at[pl.ds(r0, NPT)], oute_hbm.at[c].at[pl.ds(r0, NPT)])

  return sc_agg


def _tc_reshape(h, N, D, DH):
  B = 400

  def body(h_ref, o_ref):
    hb = h_ref[...]
    o_ref[0] = hb[:, :DH]
    o_ref[1] = hb[:, DH:]

  return pl.pallas_call(
      body,
      grid=(N // B,),
      in_specs=[pl.BlockSpec((B, D), lambda i: (i, 0))],
      out_specs=pl.BlockSpec((_NC, B, DH), lambda i: (0, i, 0)),
      out_shape=jax.ShapeDtypeStruct((_NC, N, DH), jnp.float32),
  )(h)


def _tc_final(h, ph, pe, W, elast, N, D, DE, DOUT, DH):
  B = 400

  def body(h_ref, ph_ref, pe_ref, w_ref, el_ref, o_ref):
    hb = h_ref[...]
    ah0 = ph_ref[0]
    ah1 = ph_ref[1]
    ae = pe_ref[0]
    cnt = pe_ref[1][:, 0:1]
    inv = 1.0 / jnp.maximum(cnt, 1.0)
    iso = cnt == 0.0
    w1 = w_ref[:D]
    w2 = w_ref[D:2 * D]
    w3 = w_ref[2 * D:]
    dot = functools.partial(jnp.dot, preferred_element_type=jnp.float32)
    base = dot(hb, w1)
    agg = (dot(ah0, w2[:DH]) + dot(ah1, w2[DH:]) + dot(ae, w3)) * inv
    iso_mm = dot(hb, w2) + dot(el_ref[0:1, :], w3)
    out = base + jnp.where(iso, iso_mm, agg)
    o_ref[...] = jnp.where(out > 0.0, out, jnp.exp(out) - 1.0)

  return pl.pallas_call(
      body,
      grid=(N // B,),
      in_specs=[
          pl.BlockSpec((B, D), lambda i: (i, 0)),
          pl.BlockSpec((_NC, B, DH), lambda i: (0, i, 0)),
          pl.BlockSpec((_NC, B, _EW), lambda i: (0, i, 0)),
          pl.BlockSpec((2 * D + DE, DOUT), lambda i: (0, 0)),
          pl.BlockSpec((8, DE), lambda i: (0, 0)),
      ],
      out_specs=pl.BlockSpec((B, DOUT), lambda i: (i, 0)),
      out_shape=jax.ShapeDtypeStruct((N, DOUT), jnp.float32),
  )(h, ph, pe, W, elast)


def kernel(h, edge_index, edge_attr, W):
  N, D = h.shape
  E = edge_index.shape[1]
  DE = edge_attr.shape[1]
  DOUT = W.shape[1]
  DH = D // _NC

  ept = E // _NS
  nch = ept // _C
  npt = -(-(N // _NS) // 8) * 8
  np_ = npt * _NS

  dst_r = edge_index[0].reshape(_NS, nch, _C)
  ones_c = jnp.concatenate(
      [jnp.ones((_C, 1), jnp.float32),
       jnp.zeros((_C, _EW - 1), jnp.float32)], axis=1)
  h_view = _tc_reshape(h, N, D, DH)
  zh = jnp.zeros((npt, DH), jnp.float32)
  ze = jnp.zeros((npt, _EW), jnp.float32)

  ph, pe = _sc_agg(np_, DH, nch, npt, DE)(
      edge_index, dst_r, edge_attr, ones_c, h_view, zh, ze)

  elast = jnp.broadcast_to(edge_attr[-1], (8, DE))
  return _tc_final(h, ph, pe, W, elast, N, D, DE, DOUT, DH)


# speedup vs baseline: 1.0684x; 1.0684x over previous
import functools

import jax
import jax.numpy as jnp
from jax import lax
from jax.experimental import pallas as pl
from jax.experimental.pallas import tpu as pltpu
from jax.experimental.pallas import tpu_sc as plsc

_NC = 2
_NS = 16
_C = 80
_EW = 16


def _sc_agg(NP, DH, NCH, NPT, DE):
  mesh = plsc.VectorSubcoreMesh(core_axis_name="c", subcore_axis_name="s")
  G = NCH // 2

  @functools.partial(
      pl.kernel,
      mesh=mesh,
      compiler_params=pltpu.CompilerParams(use_tc_tiling_on_sc=False),
      out_type=[
          jax.ShapeDtypeStruct((_NC, NP, DH), jnp.float32),
          jax.ShapeDtypeStruct((_NC, NP, _EW), jnp.float32),
      ],
      scratch_types=[
          pltpu.VMEM((NCH * _C,), jnp.int32),
          pltpu.VMEM((NCH * _C,), jnp.int32),
          pltpu.VMEM((_C, DH), jnp.float32),
          pltpu.VMEM((_C, DH), jnp.float32),
          pltpu.VMEM((_C, _EW), jnp.float32),
          pltpu.VMEM((_C, _EW), jnp.float32),
          pltpu.VMEM_SHARED((NP, DH), jnp.float32),
          pltpu.VMEM_SHARED((NP, _EW), jnp.float32),
          pltpu.SemaphoreType.DMA,
          pltpu.SemaphoreType.DMA,
          pltpu.SemaphoreType.DMA,
          pltpu.SemaphoreType.DMA,
      ],
  )
  def sc_agg(src_hbm, ea_hbm, ones_hbm, h_hbm, zh_hbm, ze_hbm,
             outh_hbm, oute_hbm,
             idx_s, idx_d, rows_v0, rows_v1, ea_v0, ea_v1,
             acc_h, acc_e, semA, semB, semC, semD):
    c = lax.axis_index("c")
    s = lax.axis_index("s")
    r0 = s * NPT
    hc = h_hbm
    pltpu.sync_copy(zh_hbm, acc_h.at[pl.ds(r0, NPT)])
    pltpu.sync_copy(ze_hbm, acc_e.at[pl.ds(r0, NPT)])
    pltpu.sync_copy(src_hbm.at[1].at[pl.ds(s * NCH * _C, NCH * _C)], idx_s)
    pltpu.sync_copy(src_hbm.at[0].at[pl.ds(s * NCH * _C, NCH * _C)], idx_d)
    def fix(k, carry):
      sl = pl.ds(k * 16, 16)
      idx_s[sl] = idx_s[sl] * 2 + c
      return carry

    lax.fori_loop(0, NCH * _C // 16, fix, 0)
    plsc.subcore_barrier()

    pltpu.async_copy(hc.at[idx_s.at[pl.ds(0, _C)]], rows_v0, semA)

    eb = s * NCH * _C

    @pl.when(c == 0)
    def _():
      pltpu.async_copy(ea_hbm.at[pl.ds(eb, _C)], ea_v0, semC)

    @pl.when(c == 1)
    def _():
      pltpu.sync_copy(ones_hbm, ea_v0)
      pltpu.sync_copy(ones_hbm, ea_v1)

    zh80 = zh_hbm.at[pl.ds(0, _C)]
    ze80 = ze_hbm.at[pl.ds(0, _C)]

    def body(g, carry):
      j0 = 2 * g
      j1 = j0 + 1
      pltpu.async_copy(hc.at[idx_s.at[pl.ds(j1 * _C, _C)]], rows_v1, semB)

      @pl.when(c == 0)
      def _():
        pltpu.async_copy(ea_hbm.at[pl.ds(eb + j1 * _C, _C)], ea_v1, semD)

      pltpu.make_async_copy(zh80, rows_v0, semA).wait()
      pltpu.sync_copy(rows_v0, acc_h.at[idx_d.at[pl.ds(j0 * _C, _C)]],
                      add=True)

      @pl.when(c == 0)
      def _():
        pltpu.make_async_copy(ze80, ea_v0, semC).wait()

      pltpu.sync_copy(ea_v0, acc_e.at[idx_d.at[pl.ds(j0 * _C, _C)]], add=True)

      @pl.when(g < G - 1)
      def _():
        pltpu.async_copy(hc.at[idx_s.at[pl.ds((j0 + 2) * _C, _C)]], rows_v0,
                         semA)

      @pl.when(jnp.logical_and(g < G - 1, c == 0))
      def _():
        pltpu.async_copy(ea_hbm.at[pl.ds(eb + (j0 + 2) * _C, _C)], ea_v0, semC)

      pltpu.make_async_copy(zh80, rows_v1, semB).wait()
      pltpu.sync_copy(rows_v1, acc_h.at[idx_d.at[pl.ds(j1 * _C, _C)]],
                      add=True)

      @pl.when(c == 0)
      def _():
        pltpu.make_async_copy(ze80, ea_v1, semD).wait()

      pltpu.sync_copy(ea_v1, acc_e.at[idx_d.at[pl.ds(j1 * _C, _C)]], add=True)
      return carry

    lax.fori_loop(0, G, body, 0)
    plsc.subcore_barrier()
    pltpu.sync_copy(acc_h.at[pl.ds(r0, NPT)], outh_hbm.at[c].at[pl.ds(r0, NPT)])
    pltpu.sync_copy(acc_e.at[pl.ds(r0, NPT)], oute_hbm.at[c].at[pl.ds(r0, NPT)])

  return sc_agg


def _tc_final(h, ph, pe, W, elast, N, D, DE, DOUT, DH):
  B = 400

  def body(h_ref, ph_ref, pe_ref, w_ref, el_ref, o_ref):
    hb = h_ref[...]
    ah0 = ph_ref[0]
    ah1 = ph_ref[1]
    ae = pe_ref[0]
    cnt = pe_ref[1][:, 0:1]
    inv = 1.0 / jnp.maximum(cnt, 1.0)
    iso = cnt == 0.0
    w1 = w_ref[:D]
    w2 = w_ref[D:2 * D]
    w3 = w_ref[2 * D:]
    dot = functools.partial(jnp.dot, preferred_element_type=jnp.float32)
    base = dot(hb, w1)
    agg = (dot(ah0, w2[:DH]) + dot(ah1, w2[DH:]) + dot(ae, w3)) * inv
    iso_mm = dot(hb, w2) + dot(el_ref[0:1, :], w3)
    out = base + jnp.where(iso, iso_mm, agg)
    o_ref[...] = jnp.where(out > 0.0, out, jnp.exp(out) - 1.0)

  return pl.pallas_call(
      body,
      grid=(N // B,),
      in_specs=[
          pl.BlockSpec((B, D), lambda i: (i, 0)),
          pl.BlockSpec((_NC, B, DH), lambda i: (0, i, 0)),
          pl.BlockSpec((_NC, B, _EW), lambda i: (0, i, 0)),
          pl.BlockSpec((2 * D + DE, DOUT), lambda i: (0, 0)),
          pl.BlockSpec((8, DE), lambda i: (0, 0)),
      ],
      out_specs=pl.BlockSpec((B, DOUT), lambda i: (i, 0)),
      out_shape=jax.ShapeDtypeStruct((N, DOUT), jnp.float32),
  )(h, ph, pe, W, elast)


def kernel(h, edge_index, edge_attr, W):
  N, D = h.shape
  E = edge_index.shape[1]
  DE = edge_attr.shape[1]
  DOUT = W.shape[1]
  DH = D // _NC

  ept = E // _NS
  nch = ept // _C
  npt = -(-(N // _NS) // 8) * 8
  np_ = npt * _NS

  ones_c = jnp.concatenate(
      [jnp.ones((_C, 1), jnp.float32),
       jnp.zeros((_C, _EW - 1), jnp.float32)], axis=1)
  h_view = h.reshape(_NC * N, DH)
  zh = jnp.zeros((npt, DH), jnp.float32)
  ze = jnp.zeros((npt, _EW), jnp.float32)

  ph, pe = _sc_agg(np_, DH, nch, npt, DE)(
      edge_index, edge_attr, ones_c, h_view, zh, ze)

  elast = jnp.broadcast_to(edge_attr[-1], (8, DE))
  return _tc_final(h, ph, pe, W, elast, N, D, DE, DOUT, DH)


# speedup vs baseline: 1.0694x; 1.0009x over previous
import functools

import jax
import jax.numpy as jnp
from jax import lax
from jax.experimental import pallas as pl
from jax.experimental.pallas import tpu as pltpu
from jax.experimental.pallas import tpu_sc as plsc

_NC = 2
_NS = 16
_C = 80
_EW = 16


def _sc_edges(NP, NCH, NPT):
  mesh = plsc.VectorSubcoreMesh(core_axis_name="c", subcore_axis_name="s")
  G = NCH // 2

  @functools.partial(
      pl.kernel,
      mesh=mesh,
      compiler_params=pltpu.CompilerParams(use_tc_tiling_on_sc=False),
      out_type=jax.ShapeDtypeStruct((_NC, NP, _EW), jnp.float32),
      scratch_types=[
          pltpu.VMEM((NCH * _C,), jnp.int32),
          pltpu.VMEM((_C, _EW), jnp.float32),
          pltpu.VMEM((_C, _EW), jnp.float32),
          pltpu.VMEM_SHARED((NP, _EW), jnp.float32),
          pltpu.SemaphoreType.DMA,
          pltpu.SemaphoreType.DMA,
      ],
  )
  def sc_edges(ei_hbm, ea_hbm, ones_hbm, ze_hbm, oute_hbm,
               idx_d, ea_v0, ea_v1, acc_e, semC, semD):
    c = lax.axis_index("c")
    s = lax.axis_index("s")
    r0 = s * NPT
    eb = s * NCH * _C
    pltpu.sync_copy(ze_hbm, acc_e.at[pl.ds(r0, NPT)])
    pltpu.sync_copy(ei_hbm.at[0].at[pl.ds(eb, NCH * _C)], idx_d)
    plsc.subcore_barrier()

    @pl.when(c == 0)
    def _():
      pltpu.async_copy(ea_hbm.at[pl.ds(eb, _C)], ea_v0, semC)

    @pl.when(c == 1)
    def _():
      pltpu.sync_copy(ones_hbm, ea_v0)
      pltpu.sync_copy(ones_hbm, ea_v1)

    ze80 = ze_hbm.at[pl.ds(0, _C)]

    def body(g, carry):
      j0 = 2 * g
      j1 = j0 + 1

      @pl.when(c == 0)
      def _():
        pltpu.async_copy(ea_hbm.at[pl.ds(eb + j1 * _C, _C)], ea_v1, semD)
        pltpu.make_async_copy(ze80, ea_v0, semC).wait()

      pltpu.sync_copy(ea_v0, acc_e.at[idx_d.at[pl.ds(j0 * _C, _C)]], add=True)

      @pl.when(jnp.logical_and(g < G - 1, c == 0))
      def _():
        pltpu.async_copy(ea_hbm.at[pl.ds(eb + (j0 + 2) * _C, _C)], ea_v0, semC)

      @pl.when(c == 0)
      def _():
        pltpu.make_async_copy(ze80, ea_v1, semD).wait()

      pltpu.sync_copy(ea_v1, acc_e.at[idx_d.at[pl.ds(j1 * _C, _C)]], add=True)
      return carry

    lax.fori_loop(0, G, body, 0)
    plsc.subcore_barrier()
    pltpu.sync_copy(acc_e.at[pl.ds(r0, NPT)], oute_hbm.at[c].at[pl.ds(r0, NPT)])

  return sc_edges


def _sc_hagg(NP, DH, NCH, NPT):
  mesh = plsc.VectorSubcoreMesh(core_axis_name="c", subcore_axis_name="s")
  G = NCH // 2

  @functools.partial(
      pl.kernel,
      mesh=mesh,
      compiler_params=pltpu.CompilerParams(use_tc_tiling_on_sc=False),
      out_type=jax.ShapeDtypeStruct((_NC, NP, DH), jnp.float32),
      scratch_types=[
          pltpu.VMEM((NCH * _C,), jnp.int32),
          pltpu.VMEM((NCH * _C,), jnp.int32),
          pltpu.VMEM((_C, DH), jnp.float32),
          pltpu.VMEM((_C, DH), jnp.float32),
          pltpu.VMEM_SHARED((NP, DH), jnp.float32),
          pltpu.SemaphoreType.DMA,
          pltpu.SemaphoreType.DMA,
      ],
  )
  def sc_hagg(ei_hbm, h_hbm, zh_hbm, outh_hbm,
              idx_s, idx_d, rows_v0, rows_v1, acc_h, semA, semB):
    c = lax.axis_index("c")
    s = lax.axis_index("s")
    r0 = s * NPT
    eb = s * NCH * _C
    pltpu.sync_copy(zh_hbm, acc_h.at[pl.ds(r0, NPT)])
    pltpu.sync_copy(ei_hbm.at[1].at[pl.ds(eb, NCH * _C)], idx_s)
    pltpu.sync_copy(ei_hbm.at[0].at[pl.ds(eb, NCH * _C)], idx_d)

    def fix(k, carry):
      sl = pl.ds(k * 16, 16)
      idx_s[sl] = idx_s[sl] * 2 + c
      return carry

    lax.fori_loop(0, NCH * _C // 16, fix, 0)
    plsc.subcore_barrier()

    pltpu.async_copy(h_hbm.at[idx_s.at[pl.ds(0, _C)]], rows_v0, semA)
    zh80 = zh_hbm.at[pl.ds(0, _C)]

    def body(g, carry):
      j0 = 2 * g
      j1 = j0 + 1
      pltpu.async_copy(h_hbm.at[idx_s.at[pl.ds(j1 * _C, _C)]], rows_v1, semB)
      pltpu.make_async_copy(zh80, rows_v0, semA).wait()
      pltpu.sync_copy(rows_v0, acc_h.at[idx_d.at[pl.ds(j0 * _C, _C)]],
                      add=True)

      @pl.when(g < G - 1)
      def _():
        pltpu.async_copy(h_hbm.at[idx_s.at[pl.ds((j0 + 2) * _C, _C)]],
                         rows_v0, semA)

      pltpu.make_async_copy(zh80, rows_v1, semB).wait()
      pltpu.sync_copy(rows_v1, acc_h.at[idx_d.at[pl.ds(j1 * _C, _C)]],
                      add=True)
      return carry

    lax.fori_loop(0, G, body, 0)
    plsc.subcore_barrier()
    pltpu.sync_copy(acc_h.at[pl.ds(r0, NPT)], outh_hbm.at[c].at[pl.ds(r0, NPT)])

  return sc_hagg


def _tc_final(h, ph, pe, W, elast, N, D, DE, DOUT, DH):
  B = 400

  def body(h_ref, ph_ref, pe_ref, w_ref, el_ref, o_ref):
    hb = h_ref[...]
    ah0 = ph_ref[0]
    ah1 = ph_ref[1]
    ae = pe_ref[0]
    cnt = pe_ref[1][:, 0:1]
    inv = 1.0 / jnp.maximum(cnt, 1.0)
    iso = cnt == 0.0
    w1 = w_ref[:D]
    w2 = w_ref[D:2 * D]
    w3 = w_ref[2 * D:]
    dot = functools.partial(jnp.dot, preferred_element_type=jnp.float32)
    base = dot(hb, w1)
    agg = (dot(ah0, w2[:DH]) + dot(ah1, w2[DH:]) + dot(ae, w3)) * inv
    iso_mm = dot(hb, w2) + dot(el_ref[0:1, :], w3)
    out = base + jnp.where(iso, iso_mm, agg)
    o_ref[...] = jnp.where(out > 0.0, out, jnp.exp(out) - 1.0)

  return pl.pallas_call(
      body,
      grid=(N // B,),
      in_specs=[
          pl.BlockSpec((B, D), lambda i: (i, 0)),
          pl.BlockSpec((_NC, B, DH), lambda i: (0, i, 0)),
          pl.BlockSpec((_NC, B, _EW), lambda i: (0, i, 0)),
          pl.BlockSpec((2 * D + DE, DOUT), lambda i: (0, 0)),
          pl.BlockSpec((8, DE), lambda i: (0, 0)),
      ],
      out_specs=pl.BlockSpec((B, DOUT), lambda i: (i, 0)),
      out_shape=jax.ShapeDtypeStruct((N, DOUT), jnp.float32),
  )(h, ph, pe, W, elast)


def kernel(h, edge_index, edge_attr, W):
  N, D = h.shape
  E = edge_index.shape[1]
  DE = edge_attr.shape[1]
  DOUT = W.shape[1]
  DH = D // _NC

  ept = E // _NS
  nch = ept // _C
  npt = -(-(N // _NS) // 8) * 8
  np_ = npt * _NS

  ones_c = jnp.concatenate(
      [jnp.ones((_C, 1), jnp.float32),
       jnp.zeros((_C, _EW - 1), jnp.float32)], axis=1)
  h_view = h.reshape(_NC * N, DH)
  zh = jnp.zeros((npt, DH), jnp.float32)
  ze = jnp.zeros((npt, _EW), jnp.float32)

  pe = _sc_edges(np_, nch, npt)(edge_index, edge_attr, ones_c, ze)
  ph = _sc_hagg(np_, DH, nch, npt)(edge_index, h_view, zh)

  elast = jnp.broadcast_to(edge_attr[-1], (8, DE))
  return _tc_final(h, ph, pe, W, elast, N, D, DE, DOUT, DH)


# speedup vs baseline: 1.1576x; 1.0825x over previous
import functools

import jax
import jax.numpy as jnp
from jax import lax
from jax.experimental import pallas as pl
from jax.experimental.pallas import tpu as pltpu
from jax.experimental.pallas import tpu_sc as plsc

_NC = 2
_NS = 16
_C = 80
_EW = 16


def _sc_edges(NP, EPT, NPT):
  mesh = plsc.VectorSubcoreMesh(core_axis_name="c", subcore_axis_name="s")
  CA = 128
  NF = EPT // CA
  TAIL = EPT - NF * CA
  G = NF // 2

  @functools.partial(
      pl.kernel,
      mesh=mesh,
      compiler_params=pltpu.CompilerParams(use_tc_tiling_on_sc=False),
      out_type=jax.ShapeDtypeStruct((_NC, NP, _EW), jnp.float32),
      scratch_types=[
          pltpu.VMEM((EPT,), jnp.int32),
          pltpu.VMEM((CA, _EW), jnp.float32),
          pltpu.VMEM((CA, _EW), jnp.float32),
          pltpu.VMEM_SHARED((NP, _EW), jnp.float32),
          pltpu.SemaphoreType.DMA,
          pltpu.SemaphoreType.DMA,
      ],
  )
  def sc_edges(ei_hbm, ea_hbm, ones_hbm, ze_hbm, oute_hbm,
               idx_d, ea_v0, ea_v1, acc_e, semC, semD):
    c = lax.axis_index("c")
    s = lax.axis_index("s")
    r0 = s * NPT
    eb = s * EPT
    pltpu.sync_copy(ze_hbm, acc_e.at[pl.ds(r0, NPT)])
    pltpu.sync_copy(ei_hbm.at[0].at[pl.ds(eb, EPT)], idx_d)
    plsc.subcore_barrier()

    @pl.when(c == 0)
    def _():
      pltpu.async_copy(ea_hbm.at[pl.ds(eb, CA)], ea_v0, semC)

    @pl.when(c == 1)
    def _():
      pltpu.sync_copy(ones_hbm, ea_v0)
      pltpu.sync_copy(ones_hbm, ea_v1)

    zeCA = ze_hbm.at[pl.ds(0, CA)]

    def body(g, carry):
      j0 = 2 * g
      j1 = j0 + 1

      @pl.when(c == 0)
      def _():
        pltpu.async_copy(ea_hbm.at[pl.ds(eb + j1 * CA, CA)], ea_v1, semD)
        pltpu.make_async_copy(zeCA, ea_v0, semC).wait()

      pltpu.sync_copy(ea_v0, acc_e.at[idx_d.at[pl.ds(j0 * CA, CA)]], add=True)

      @pl.when(jnp.logical_and(g < G - 1, c == 0))
      def _():
        pltpu.async_copy(ea_hbm.at[pl.ds(eb + (j0 + 2) * CA, CA)], ea_v0, semC)

      @pl.when(c == 0)
      def _():
        pltpu.make_async_copy(zeCA, ea_v1, semD).wait()

      pltpu.sync_copy(ea_v1, acc_e.at[idx_d.at[pl.ds(j1 * CA, CA)]], add=True)
      return carry

    lax.fori_loop(0, G, body, 0)

    tb = NF * CA

    @pl.when(c == 0)
    def _():
      pltpu.sync_copy(ea_hbm.at[pl.ds(eb + tb, TAIL)], ea_v0.at[pl.ds(0, TAIL)])

    pltpu.sync_copy(ea_v0.at[pl.ds(0, TAIL)],
                    acc_e.at[idx_d.at[pl.ds(tb, TAIL)]], add=True)
    plsc.subcore_barrier()
    pltpu.sync_copy(acc_e.at[pl.ds(r0, NPT)], oute_hbm.at[c].at[pl.ds(r0, NPT)])

  return sc_edges


def _sc_hagg(NP, DH, NCH, NPT):
  mesh = plsc.VectorSubcoreMesh(core_axis_name="c", subcore_axis_name="s")
  G = NCH // 2

  @functools.partial(
      pl.kernel,
      mesh=mesh,
      compiler_params=pltpu.CompilerParams(use_tc_tiling_on_sc=False),
      out_type=jax.ShapeDtypeStruct((_NC, NP, DH), jnp.float32),
      scratch_types=[
          pltpu.VMEM((NCH * _C,), jnp.int32),
          pltpu.VMEM((NCH * _C,), jnp.int32),
          pltpu.VMEM((_C, DH), jnp.float32),
          pltpu.VMEM((_C, DH), jnp.float32),
          pltpu.VMEM_SHARED((NP, DH), jnp.float32),
          pltpu.SemaphoreType.DMA,
          pltpu.SemaphoreType.DMA,
      ],
  )
  def sc_hagg(ei_hbm, h_hbm, zh_hbm, outh_hbm,
              idx_s, idx_d, rows_v0, rows_v1, acc_h, semA, semB):
    c = lax.axis_index("c")
    s = lax.axis_index("s")
    r0 = s * NPT
    eb = s * NCH * _C
    pltpu.sync_copy(zh_hbm, acc_h.at[pl.ds(r0, NPT)])
    pltpu.sync_copy(ei_hbm.at[1].at[pl.ds(eb, NCH * _C)], idx_s)
    pltpu.sync_copy(ei_hbm.at[0].at[pl.ds(eb, NCH * _C)], idx_d)

    def fix(k, carry):
      sl = pl.ds(k * 16, 16)
      idx_s[sl] = idx_s[sl] * 2 + c
      return carry

    lax.fori_loop(0, NCH * _C // 16, fix, 0)
    plsc.subcore_barrier()

    pltpu.async_copy(h_hbm.at[idx_s.at[pl.ds(0, _C)]], rows_v0, semA)
    zh80 = zh_hbm.at[pl.ds(0, _C)]

    def body(g, carry):
      j0 = 2 * g
      j1 = j0 + 1
      pltpu.async_copy(h_hbm.at[idx_s.at[pl.ds(j1 * _C, _C)]], rows_v1, semB)
      pltpu.make_async_copy(zh80, rows_v0, semA).wait()
      pltpu.sync_copy(rows_v0, acc_h.at[idx_d.at[pl.ds(j0 * _C, _C)]],
                      add=True)

      @pl.when(g < G - 1)
      def _():
        pltpu.async_copy(h_hbm.at[idx_s.at[pl.ds((j0 + 2) * _C, _C)]],
                         rows_v0, semA)

      pltpu.make_async_copy(zh80, rows_v1, semB).wait()
      pltpu.sync_copy(rows_v1, acc_h.at[idx_d.at[pl.ds(j1 * _C, _C)]],
                      add=True)
      return carry

    lax.fori_loop(0, G, body, 0)
    plsc.subcore_barrier()
    pltpu.sync_copy(acc_h.at[pl.ds(r0, NPT)], outh_hbm.at[c].at[pl.ds(r0, NPT)])

  return sc_hagg


def _tc_final(h, ph, pe, W, elast, N, D, DE, DOUT, DH):
  B = 400

  def body(h_ref, ph_ref, pe_ref, w_ref, el_ref, o_ref):
    hb = h_ref[...]
    ah0 = ph_ref[0]
    ah1 = ph_ref[1]
    ae = pe_ref[0]
    cnt = pe_ref[1][:, 0:1]
    inv = 1.0 / jnp.maximum(cnt, 1.0)
    iso = cnt == 0.0
    w1 = w_ref[:D]
    w2 = w_ref[D:2 * D]
    w3 = w_ref[2 * D:]
    dot = functools.partial(jnp.dot, preferred_element_type=jnp.float32)
    base = dot(hb, w1)
    agg = (dot(ah0, w2[:DH]) + dot(ah1, w2[DH:]) + dot(ae, w3)) * inv
    iso_mm = dot(hb, w2) + dot(el_ref[0:1, :], w3)
    out = base + jnp.where(iso, iso_mm, agg)
    o_ref[...] = jnp.where(out > 0.0, out, jnp.exp(out) - 1.0)

  return pl.pallas_call(
      body,
      grid=(N // B,),
      in_specs=[
          pl.BlockSpec((B, D), lambda i: (i, 0)),
          pl.BlockSpec((_NC, B, DH), lambda i: (0, i, 0)),
          pl.BlockSpec((_NC, B, _EW), lambda i: (0, i, 0)),
          pl.BlockSpec((2 * D + DE, DOUT), lambda i: (0, 0)),
          pl.BlockSpec((8, DE), lambda i: (0, 0)),
      ],
      out_specs=pl.BlockSpec((B, DOUT), lambda i: (i, 0)),
      out_shape=jax.ShapeDtypeStruct((N, DOUT), jnp.float32),
  )(h, ph, pe, W, elast)


def kernel(h, edge_index, edge_attr, W):
  N, D = h.shape
  E = edge_index.shape[1]
  DE = edge_attr.shape[1]
  DOUT = W.shape[1]
  DH = D // _NC

  ept = E // _NS
  nch = ept // _C
  npt = -(-(N // _NS) // 8) * 8
  np_ = npt * _NS

  ones_c = jnp.concatenate(
      [jnp.ones((128, 1), jnp.float32),
       jnp.zeros((128, _EW - 1), jnp.float32)], axis=1)
  h_view = h.reshape(_NC * N, DH)
  zh = jnp.zeros((npt, DH), jnp.float32)
  ze = jnp.zeros((npt, _EW), jnp.float32)

  pe = _sc_edges(np_, ept, npt)(edge_index, edge_attr, ones_c, ze)
  ph = _sc_hagg(np_, DH, nch, npt)(edge_index, h_view, zh)

  elast = jnp.broadcast_to(edge_attr[-1], (8, DE))
  return _tc_final(h, ph, pe, W, elast, N, D, DE, DOUT, DH)


# speedup vs baseline: 1.2606x; 1.0890x over previous
import functools

import jax
import jax.numpy as jnp
from jax import lax
from jax.experimental import pallas as pl
from jax.experimental.pallas import tpu as pltpu
from jax.experimental.pallas import tpu_sc as plsc

_NC = 2
_NS = 16
_C = 80
_EW = 16


def _sc_edges(NP, EPT, NPT):
  mesh = plsc.VectorSubcoreMesh(core_axis_name="c", subcore_axis_name="s")
  CA = 128
  NF = EPT // CA
  TAIL = EPT - NF * CA
  G = NF // 2

  @functools.partial(
      pl.kernel,
      mesh=mesh,
      compiler_params=pltpu.CompilerParams(use_tc_tiling_on_sc=False),
      out_type=jax.ShapeDtypeStruct((_NC, NP, _EW), jnp.float32),
      scratch_types=[
          pltpu.VMEM((EPT,), jnp.int32),
          pltpu.VMEM((CA, _EW), jnp.float32),
          pltpu.VMEM((CA, _EW), jnp.float32),
          pltpu.VMEM_SHARED((NP, _EW), jnp.float32),
          pltpu.SemaphoreType.DMA,
          pltpu.SemaphoreType.DMA,
      ],
  )
  def sc_edges(ei_hbm, ea_hbm, ones_hbm, ze_hbm, oute_hbm,
               idx_d, ea_v0, ea_v1, acc_e, semC, semD):
    c = lax.axis_index("c")
    s = lax.axis_index("s")
    r0 = s * NPT
    eb = s * EPT
    pltpu.sync_copy(ze_hbm, acc_e.at[pl.ds(r0, NPT)])
    pltpu.sync_copy(ei_hbm.at[0].at[pl.ds(eb, EPT)], idx_d)
    plsc.subcore_barrier()

    @pl.when(c == 0)
    def _():
      pltpu.async_copy(ea_hbm.at[pl.ds(eb, CA)], ea_v0, semC)

    @pl.when(c == 1)
    def _():
      pltpu.sync_copy(ones_hbm, ea_v0)
      pltpu.sync_copy(ones_hbm, ea_v1)

    zeCA = ze_hbm.at[pl.ds(0, CA)]

    def body(g, carry):
      j0 = 2 * g
      j1 = j0 + 1

      @pl.when(c == 0)
      def _():
        pltpu.async_copy(ea_hbm.at[pl.ds(eb + j1 * CA, CA)], ea_v1, semD)
        pltpu.make_async_copy(zeCA, ea_v0, semC).wait()

      pltpu.sync_copy(ea_v0, acc_e.at[idx_d.at[pl.ds(j0 * CA, CA)]], add=True)

      @pl.when(jnp.logical_and(g < G - 1, c == 0))
      def _():
        pltpu.async_copy(ea_hbm.at[pl.ds(eb + (j0 + 2) * CA, CA)], ea_v0, semC)

      @pl.when(c == 0)
      def _():
        pltpu.make_async_copy(zeCA, ea_v1, semD).wait()

      pltpu.sync_copy(ea_v1, acc_e.at[idx_d.at[pl.ds(j1 * CA, CA)]], add=True)
      return carry

    lax.fori_loop(0, G, body, 0)

    tb = NF * CA

    @pl.when(c == 0)
    def _():
      pltpu.sync_copy(ea_hbm.at[pl.ds(eb + tb, TAIL)], ea_v0.at[pl.ds(0, TAIL)])

    pltpu.sync_copy(ea_v0.at[pl.ds(0, TAIL)],
                    acc_e.at[idx_d.at[pl.ds(tb, TAIL)]], add=True)
    plsc.subcore_barrier()
    pltpu.sync_copy(acc_e.at[pl.ds(r0, NPT)], oute_hbm.at[c].at[pl.ds(r0, NPT)])

  return sc_edges


def _sc_hagg(NP, DH, EPT, NPT):
  mesh = plsc.VectorSubcoreMesh(core_axis_name="c", subcore_axis_name="s")
  CB = 128
  NF = EPT // CB
  TAIL = EPT - NF * CB
  G = NF // 2

  @functools.partial(
      pl.kernel,
      mesh=mesh,
      compiler_params=pltpu.CompilerParams(use_tc_tiling_on_sc=False),
      out_type=jax.ShapeDtypeStruct((_NC, NP, DH), jnp.float32),
      scratch_types=[
          pltpu.VMEM((EPT,), jnp.int32),
          pltpu.VMEM((EPT,), jnp.int32),
          pltpu.VMEM((CB, DH), jnp.float32),
          pltpu.VMEM((CB, DH), jnp.float32),
          pltpu.VMEM_SHARED((NP, DH), jnp.float32),
          pltpu.SemaphoreType.DMA,
          pltpu.SemaphoreType.DMA,
      ],
  )
  def sc_hagg(ei_hbm, h_hbm, zh_hbm, outh_hbm,
              idx_s, idx_d, rows_v0, rows_v1, acc_h, semA, semB):
    c = lax.axis_index("c")
    s = lax.axis_index("s")
    r0 = s * NPT
    eb = s * EPT
    pltpu.sync_copy(zh_hbm, acc_h.at[pl.ds(r0, NPT)])
    pltpu.sync_copy(ei_hbm.at[1].at[pl.ds(eb, EPT)], idx_s)
    pltpu.sync_copy(ei_hbm.at[0].at[pl.ds(eb, EPT)], idx_d)

    def fix(k, carry):
      sl = pl.ds(k * 16, 16)
      idx_s[sl] = idx_s[sl] * 2 + c
      return carry

    lax.fori_loop(0, EPT // 16, fix, 0)
    plsc.subcore_barrier()

    pltpu.async_copy(h_hbm.at[idx_s.at[pl.ds(0, CB)]], rows_v0, semA)
    zhCB = zh_hbm.at[pl.ds(0, CB)]

    def body(g, carry):
      j0 = 2 * g
      j1 = j0 + 1
      pltpu.async_copy(h_hbm.at[idx_s.at[pl.ds(j1 * CB, CB)]], rows_v1, semB)
      pltpu.make_async_copy(zhCB, rows_v0, semA).wait()
      pltpu.sync_copy(rows_v0, acc_h.at[idx_d.at[pl.ds(j0 * CB, CB)]],
                      add=True)

      @pl.when(g < G - 1)
      def _():
        pltpu.async_copy(h_hbm.at[idx_s.at[pl.ds((j0 + 2) * CB, CB)]],
                         rows_v0, semA)

      pltpu.make_async_copy(zhCB, rows_v1, semB).wait()
      pltpu.sync_copy(rows_v1, acc_h.at[idx_d.at[pl.ds(j1 * CB, CB)]],
                      add=True)
      return carry

    lax.fori_loop(0, G, body, 0)

    tb = NF * CB
    pltpu.async_copy(h_hbm.at[idx_s.at[pl.ds(tb, TAIL)]],
                     rows_v0.at[pl.ds(0, TAIL)], semA)
    pltpu.make_async_copy(zh_hbm.at[pl.ds(0, TAIL)],
                          rows_v0.at[pl.ds(0, TAIL)], semA).wait()
    pltpu.sync_copy(rows_v0.at[pl.ds(0, TAIL)],
                    acc_h.at[idx_d.at[pl.ds(tb, TAIL)]], add=True)
    plsc.subcore_barrier()
    pltpu.sync_copy(acc_h.at[pl.ds(r0, NPT)], outh_hbm.at[c].at[pl.ds(r0, NPT)])

  return sc_hagg


def _tc_final(h, ph, pe, W, elast, N, D, DE, DOUT, DH):
  B = 400

  def body(h_ref, ph_ref, pe_ref, w_ref, el_ref, o_ref):
    hb = h_ref[...]
    ah0 = ph_ref[0]
    ah1 = ph_ref[1]
    ae = pe_ref[0]
    cnt = pe_ref[1][:, 0:1]
    inv = 1.0 / jnp.maximum(cnt, 1.0)
    iso = cnt == 0.0
    w1 = w_ref[:D]
    w2 = w_ref[D:2 * D]
    w3 = w_ref[2 * D:]
    dot = functools.partial(jnp.dot, preferred_element_type=jnp.float32)
    base = dot(hb, w1)
    agg = (dot(ah0, w2[:DH]) + dot(ah1, w2[DH:]) + dot(ae, w3)) * inv
    iso_mm = dot(hb, w2) + dot(el_ref[0:1, :], w3)
    out = base + jnp.where(iso, iso_mm, agg)
    o_ref[...] = jnp.where(out > 0.0, out, jnp.exp(out) - 1.0)

  return pl.pallas_call(
      body,
      grid=(N // B,),
      in_specs=[
          pl.BlockSpec((B, D), lambda i: (i, 0)),
          pl.BlockSpec((_NC, B, DH), lambda i: (0, i, 0)),
          pl.BlockSpec((_NC, B, _EW), lambda i: (0, i, 0)),
          pl.BlockSpec((2 * D + DE, DOUT), lambda i: (0, 0)),
          pl.BlockSpec((8, DE), lambda i: (0, 0)),
      ],
      out_specs=pl.BlockSpec((B, DOUT), lambda i: (i, 0)),
      out_shape=jax.ShapeDtypeStruct((N, DOUT), jnp.float32),
  )(h, ph, pe, W, elast)


def kernel(h, edge_index, edge_attr, W):
  N, D = h.shape
  E = edge_index.shape[1]
  DE = edge_attr.shape[1]
  DOUT = W.shape[1]
  DH = D // _NC

  ept = E // _NS
  nch = ept // _C
  npt = -(-(N // _NS) // 8) * 8
  np_ = npt * _NS

  ones_c = jnp.concatenate(
      [jnp.ones((128, 1), jnp.float32),
       jnp.zeros((128, _EW - 1), jnp.float32)], axis=1)
  h_view = h.reshape(_NC * N, DH)
  zh = jnp.zeros((npt, DH), jnp.float32)
  ze = jnp.zeros((npt, _EW), jnp.float32)

  pe = _sc_edges(np_, ept, npt)(edge_index, edge_attr, ones_c, ze)
  ph = _sc_hagg(np_, DH, ept, npt)(edge_index, h_view, zh)

  elast = jnp.broadcast_to(edge_attr[-1], (8, DE))
  return _tc_final(h, ph, pe, W, elast, N, D, DE, DOUT, DH)
